# baseline (speedup 1.0000x reference)
import functools

import jax
import jax.numpy as jnp
from jax import lax
from jax.experimental import pallas as pl
from jax.experimental.pallas import tpu as pltpu
from jax.experimental.pallas import tpu_sc as plsc

NC = 2
NS = 16
LANES = 16
HIMASK = -65536


@functools.lru_cache(maxsize=None)
def _make_sc_scores(B, NEG, D, C):
    NW = NC * NS
    BPW = B // NW
    NCHUNK = BPW // C
    NIDX = C * NEG
    KROWS = NIDX // 128
    DP = D // 2

    mesh = plsc.VectorSubcoreMesh(core_axis_name="c", subcore_axis_name="s")

    @functools.partial(
        pl.kernel,
        mesh=mesh,
        compiler_params=pltpu.CompilerParams(
            needs_layout_passes=False, use_tc_tiling_on_sc=False),
        out_type=[
            jax.ShapeDtypeStruct((B,), jnp.float32),
            jax.ShapeDtypeStruct((B * NEG,), jnp.float32),
        ],
        scratch_types=[
            pltpu.VMEM((C,), jnp.int32),
            pltpu.VMEM((C,), jnp.int32),
            pltpu.VMEM((NIDX,), jnp.int32),
            pltpu.VMEM((C, DP), jnp.float32),
            pltpu.VMEM((C, DP), jnp.float32),
            pltpu.VMEM((NIDX, DP), jnp.float32),
            pltpu.VMEM((C,), jnp.float32),
            pltpu.VMEM((NIDX,), jnp.float32),
            pltpu.SemaphoreType.DMA,
        ],
    )
    def sc_scores(center_hbm, pos_hbm, negr_hbm, inw_hbm, outw_hbm,
                  pos_out, neg_out,
                  idx_c, idx_p, idx_n, v_rows, p_rows, n_rows,
                  pos_s, neg_s, sem):
        wid = lax.axis_index("s") * NC + lax.axis_index("c")
        base = wid * BPW

        def chunk(ci, chunk_carry):
            off = base + ci * C
            pltpu.sync_copy(center_hbm.at[pl.ds(off, C)], idx_c)
            pltpu.sync_copy(pos_hbm.at[pl.ds(off, C)], idx_p)
            pltpu.sync_copy(negr_hbm.at[pl.ds(off * NEG, NIDX)], idx_n)
            cps = [
                pltpu.async_copy(inw_hbm.at[idx_c], v_rows, sem),
                pltpu.async_copy(outw_hbm.at[idx_p], p_rows, sem),
            ]
            for j in range(KROWS):
                cps.append(pltpu.async_copy(
                    outw_hbm.at[idx_n.at[pl.ds(j * 128, 128)]],
                    n_rows.at[pl.ds(j * 128, 128)], sem))
            for cp in cps:
                cp.wait()

            lane = lax.iota(jnp.int32, LANES)

            def rowvecs(ref, r):
                vs = []
                for j in range(DP // 16):
                    w = plsc.bitcast(ref[r, pl.ds(16 * j, 16)], jnp.int32)
                    lo = plsc.bitcast(w << 16, jnp.float32)
                    hi = plsc.bitcast(w & HIMASK, jnp.float32)
                    vs += [lo, hi]
                return vs

            def dot(vs, ref, r):
                us = rowvecs(ref, r)
                acc = vs[0] * us[0]
                for j in range(1, len(vs)):
                    acc = acc + vs[j] * us[j]
                return jnp.sum(acc)

            def pos_group(g, carry):
                acc = jnp.zeros((LANES,), jnp.float32)
                for t in range(LANES):
                    i = g * LANES + t
                    vs = rowvecs(v_rows, i)
                    s = dot(vs, p_rows, i)
                    acc = jnp.where(lane == t, s, acc)
                pos_s[pl.ds(g * LANES, LANES)] = acc
                return carry

            lax.fori_loop(0, C // LANES, pos_group, 0)

            def neg_group(g, carry):
                accs = [jnp.zeros((LANES,), jnp.float32) for _ in range(5)]
                for ai in range(4):
                    i = g * 4 + ai
                    vs = rowvecs(v_rows, i)
                    for k in range(NEG):
                        rloc = ai * NEG + k
                        s = dot(vs, n_rows, i * NEG + k)
                        accs[rloc // LANES] = jnp.where(
                            lane == rloc % LANES, s, accs[rloc // LANES])
                for m in range(5):
                    neg_s[pl.ds(g * 4 * NEG + m * LANES, LANES)] = accs[m]
                return carry

            lax.fori_loop(0, C // 4, neg_group, 0)
            pltpu.sync_copy(pos_s, pos_out.at[pl.ds(off, C)])
            pltpu.sync_copy(neg_s, neg_out.at[pl.ds(off * NEG, NIDX)])
            return chunk_carry

        lax.fori_loop(0, NCHUNK, chunk, 0)

    return sc_scores


@functools.lru_cache(maxsize=None)
def _make_tc_relayout(V, D, CB):
    NB = (V + CB - 1) // CB
    Q = CB // 4

    def body(x_ref, y_ref):
        t = x_ref[...].T
        lo = jax.lax.bitcast_convert_type(t[:, : D // 2], jnp.int32)
        hi = jax.lax.bitcast_convert_type(t[:, D // 2:], jnp.int32)
        word = (hi & HIMASK) | jax.lax.shift_right_logical(lo, 16)
        pk = jax.lax.bitcast_convert_type(word, jnp.float32)
        y_ref[...] = jnp.concatenate(
            [pk[0 * Q:1 * Q], pk[1 * Q:2 * Q],
             pk[2 * Q:3 * Q], pk[3 * Q:4 * Q]], axis=-1)

    return pl.pallas_call(
        body,
        grid=(NB,),
        in_specs=[pl.BlockSpec((D, CB), lambda i: (0, i))],
        out_specs=pl.BlockSpec((Q, 2 * D), lambda i: (i, 0)),
        out_shape=jax.ShapeDtypeStruct((NB * Q, 2 * D), jnp.float32),
    )


def _log_sigmoid(x):
    return jnp.minimum(x, 0.0) - jnp.log1p(jnp.exp(-jnp.abs(x)))


@functools.lru_cache(maxsize=None)
def _make_tc_loss(B, NEG):
    def body(pos_ref, neg_ref, out_ref):
        pos_ls = _log_sigmoid(pos_ref[...])
        neg_ls = _log_sigmoid(-neg_ref[...])
        out_ref[0, 0] = -(jnp.sum(pos_ls) + jnp.sum(neg_ls)) / B

    return pl.pallas_call(
        body,
        out_shape=jax.ShapeDtypeStruct((1, 1), jnp.float32),
        out_specs=pl.BlockSpec(memory_space=pltpu.SMEM),
    )


def kernel(center_words, pos_context, neg_context, in_weight, out_weight):
    B, NEG = neg_context.shape
    V, D = in_weight.shape
    CB = 4096
    NB = (V + CB - 1) // CB

    def remap(idx):
        idx = idx.astype(jnp.int32)
        blk = idx // CB
        rem = idx % CB
        return blk * CB + (rem % (CB // 4)) * 4 + rem // (CB // 4)

    cw = remap(center_words)
    pc = remap(pos_context)
    ncr = remap(neg_context).reshape(B * NEG)
    relayout = _make_tc_relayout(V, D, CB)
    lin1 = relayout(in_weight.T).reshape(NB * CB, D // 2)
    lin2 = relayout(out_weight.T).reshape(NB * CB, D // 2)
    pos_s, neg_s = _make_sc_scores(B, NEG, D, 32)(
        cw, pc, ncr, lin1, lin2)
    loss = _make_tc_loss(B, NEG)(
        pos_s.reshape(B // 128, 128), neg_s.reshape(B * NEG // 128, 128))
    return loss.reshape(())


# speedup vs baseline: 1.2016x; 1.2016x over previous
import functools

import jax
import jax.numpy as jnp
from jax import lax
from jax.experimental import pallas as pl
from jax.experimental.pallas import tpu as pltpu
from jax.experimental.pallas import tpu_sc as plsc

NC = 2
NS = 16
LANES = 16
HIMASK = -65536


@functools.lru_cache(maxsize=None)
def _make_sc_scores(B, NEG, D, C):
    NW = NC * NS
    BPW = B // NW
    NCHUNK = BPW // C
    NIDX = C * NEG
    KROWS = NIDX // 128
    DP = D // 2

    mesh = plsc.VectorSubcoreMesh(core_axis_name="c", subcore_axis_name="s")

    @functools.partial(
        pl.kernel,
        mesh=mesh,
        compiler_params=pltpu.CompilerParams(
            needs_layout_passes=False, use_tc_tiling_on_sc=False),
        out_type=[
            jax.ShapeDtypeStruct((B,), jnp.float32),
            jax.ShapeDtypeStruct((B * NEG,), jnp.float32),
        ],
        scratch_types=[
            pltpu.VMEM((C,), jnp.int32),
            pltpu.VMEM((C,), jnp.int32),
            pltpu.VMEM((NIDX,), jnp.int32),
            pltpu.VMEM((C, DP), jnp.float32),
            pltpu.VMEM((C, DP), jnp.float32),
            pltpu.VMEM((NIDX, DP), jnp.float32),
            pltpu.VMEM((C,), jnp.float32),
            pltpu.VMEM((NIDX,), jnp.float32),
            pltpu.SemaphoreType.DMA,
        ],
    )
    def sc_scores(center_hbm, pos_hbm, negr_hbm, inw_hbm, outw_hbm,
                  pos_out, neg_out,
                  idx_c, idx_p, idx_n, v_rows, p_rows, n_rows,
                  pos_s, neg_s, sem):
        wid = lax.axis_index("s") * NC + lax.axis_index("c")
        base = wid * BPW

        def chunk(ci, chunk_carry):
            off = base + ci * C
            pltpu.sync_copy(center_hbm.at[pl.ds(off, C)], idx_c)
            pltpu.sync_copy(pos_hbm.at[pl.ds(off, C)], idx_p)
            pltpu.sync_copy(negr_hbm.at[pl.ds(off * NEG, NIDX)], idx_n)
            cps = [
                pltpu.async_copy(inw_hbm.at[idx_c], v_rows, sem),
                pltpu.async_copy(outw_hbm.at[idx_p], p_rows, sem),
            ]
            for j in range(KROWS):
                cps.append(pltpu.async_copy(
                    outw_hbm.at[idx_n.at[pl.ds(j * 128, 128)]],
                    n_rows.at[pl.ds(j * 128, 128)], sem))
            for cp in cps:
                cp.wait()

            lane = lax.iota(jnp.int32, LANES)

            def rowvecs(ref, r):
                vs = []
                for j in range(DP // 16):
                    w = plsc.bitcast(ref[r, pl.ds(16 * j, 16)], jnp.int32)
                    lo = plsc.bitcast(w << 16, jnp.float32)
                    hi = plsc.bitcast(w & HIMASK, jnp.float32)
                    vs += [lo, hi]
                return vs

            def dot(vs, ref, r):
                us = rowvecs(ref, r)
                acc = vs[0] * us[0]
                for j in range(1, len(vs)):
                    acc = acc + vs[j] * us[j]
                return jnp.sum(acc)

            def pos_group(g, carry):
                acc = jnp.zeros((LANES,), jnp.float32)
                for t in range(LANES):
                    i = g * LANES + t
                    vs = rowvecs(v_rows, i)
                    s = dot(vs, p_rows, i)
                    acc = jnp.where(lane == t, s, acc)
                pos_s[pl.ds(g * LANES, LANES)] = acc
                return carry

            lax.fori_loop(0, C // LANES, pos_group, 0)

            def neg_group(g, carry):
                accs = [jnp.zeros((LANES,), jnp.float32) for _ in range(5)]
                for ai in range(4):
                    i = g * 4 + ai
                    vs = rowvecs(v_rows, i)
                    for k in range(NEG):
                        rloc = ai * NEG + k
                        s = dot(vs, n_rows, i * NEG + k)
                        accs[rloc // LANES] = jnp.where(
                            lane == rloc % LANES, s, accs[rloc // LANES])
                for m in range(5):
                    neg_s[pl.ds(g * 4 * NEG + m * LANES, LANES)] = accs[m]
                return carry

            lax.fori_loop(0, C // 4, neg_group, 0)
            pltpu.sync_copy(pos_s, pos_out.at[pl.ds(off, C)])
            pltpu.sync_copy(neg_s, neg_out.at[pl.ds(off * NEG, NIDX)])
            return chunk_carry

        lax.fori_loop(0, NCHUNK, chunk, 0)

    return sc_scores


@functools.lru_cache(maxsize=None)
def _make_tc_relayout(V, D, CB):
    NB = (V + CB - 1) // CB
    Q = CB // 4

    def body(x_ref, y_ref):
        x = x_ref[...]
        lo = jax.lax.bitcast_convert_type(x[: D // 2, :], jnp.int32)
        hi = jax.lax.bitcast_convert_type(x[D // 2:, :], jnp.int32)
        word = (hi & HIMASK) | jax.lax.shift_right_logical(lo, 16)
        pk = jax.lax.bitcast_convert_type(word, jnp.float32).T
        y_ref[...] = jnp.concatenate(
            [pk[0 * Q:1 * Q], pk[1 * Q:2 * Q],
             pk[2 * Q:3 * Q], pk[3 * Q:4 * Q]], axis=-1)

    return pl.pallas_call(
        body,
        grid=(NB,),
        in_specs=[pl.BlockSpec((D, CB), lambda i: (0, i))],
        out_specs=pl.BlockSpec((Q, 2 * D), lambda i: (i, 0)),
        out_shape=jax.ShapeDtypeStruct((NB * Q, 2 * D), jnp.float32),
    )


def _log_sigmoid(x):
    return jnp.minimum(x, 0.0) - jnp.log1p(jnp.exp(-jnp.abs(x)))


@functools.lru_cache(maxsize=None)
def _make_tc_loss(B, NEG):
    def body(pos_ref, neg_ref, out_ref):
        pos_ls = _log_sigmoid(pos_ref[...])
        neg_ls = _log_sigmoid(-neg_ref[...])
        out_ref[0, 0] = -(jnp.sum(pos_ls) + jnp.sum(neg_ls)) / B

    return pl.pallas_call(
        body,
        out_shape=jax.ShapeDtypeStruct((1, 1), jnp.float32),
        out_specs=pl.BlockSpec(memory_space=pltpu.SMEM),
    )


def kernel(center_words, pos_context, neg_context, in_weight, out_weight):
    B, NEG = neg_context.shape
    V, D = in_weight.shape
    CB = 4096
    NB = (V + CB - 1) // CB

    def remap(idx):
        idx = idx.astype(jnp.int32)
        blk = idx // CB
        rem = idx % CB
        return blk * CB + (rem % (CB // 4)) * 4 + rem // (CB // 4)

    cw = remap(center_words)
    pc = remap(pos_context)
    ncr = remap(neg_context).reshape(B * NEG)
    relayout = _make_tc_relayout(V, D, CB)
    lin1 = relayout(in_weight.T).reshape(NB * CB, D // 2)
    lin2 = relayout(out_weight.T).reshape(NB * CB, D // 2)
    pos_s, neg_s = _make_sc_scores(B, NEG, D, 32)(
        cw, pc, ncr, lin1, lin2)
    loss = _make_tc_loss(B, NEG)(
        pos_s.reshape(B // 128, 128), neg_s.reshape(B * NEG // 128, 128))
    return loss.reshape(())


# speedup vs baseline: 1.4427x; 1.2006x over previous
import functools

import jax
import jax.numpy as jnp
from jax import lax
from jax.experimental import pallas as pl
from jax.experimental.pallas import tpu as pltpu
from jax.experimental.pallas import tpu_sc as plsc

NC = 2
NS = 16
LANES = 16
HIMASK = -65536


@functools.lru_cache(maxsize=None)
def _make_sc_scores(B, NEG, D, C):
    NW = NC * NS
    BPW = B // NW
    NCHUNK = BPW // C
    NIDX = C * NEG
    KROWS = NIDX // 128
    DP = D // 2

    mesh = plsc.VectorSubcoreMesh(core_axis_name="c", subcore_axis_name="s")

    @functools.partial(
        pl.kernel,
        mesh=mesh,
        compiler_params=pltpu.CompilerParams(
            needs_layout_passes=False, use_tc_tiling_on_sc=False),
        out_type=[
            jax.ShapeDtypeStruct((B,), jnp.float32),
            jax.ShapeDtypeStruct((B * NEG,), jnp.float32),
        ],
        scratch_types=[
            pltpu.VMEM((C,), jnp.int32),
            pltpu.VMEM((C,), jnp.int32),
            pltpu.VMEM((NIDX,), jnp.int32),
            pltpu.VMEM((C, DP), jnp.float32),
            pltpu.VMEM((C, DP), jnp.float32),
            pltpu.VMEM((NIDX, DP), jnp.float32),
            pltpu.VMEM((C,), jnp.float32),
            pltpu.VMEM((NIDX,), jnp.float32),
            pltpu.SemaphoreType.DMA,
        ],
    )
    def sc_scores(center_hbm, pos_hbm, negr_hbm, inw_hbm, outw_hbm,
                  pos_out, neg_out,
                  idx_c, idx_p, idx_n, v_rows, p_rows, n_rows,
                  pos_s, neg_s, sem):
        wid = lax.axis_index("s") * NC + lax.axis_index("c")
        base = wid * BPW

        def chunk(ci, chunk_carry):
            off = base + ci * C
            pltpu.sync_copy(center_hbm.at[pl.ds(off, C)], idx_c)
            pltpu.sync_copy(pos_hbm.at[pl.ds(off, C)], idx_p)
            pltpu.sync_copy(negr_hbm.at[pl.ds(off * NEG, NIDX)], idx_n)
            cps = [
                pltpu.async_copy(inw_hbm.at[idx_c], v_rows, sem),
                pltpu.async_copy(outw_hbm.at[idx_p], p_rows, sem),
            ]
            for j in range(KROWS):
                cps.append(pltpu.async_copy(
                    outw_hbm.at[idx_n.at[pl.ds(j * 128, 128)]],
                    n_rows.at[pl.ds(j * 128, 128)], sem))
            for cp in cps:
                cp.wait()

            lane = lax.iota(jnp.int32, LANES)

            def rowvecs(ref, r):
                vs = []
                for j in range(DP // 16):
                    w = plsc.bitcast(ref[r, pl.ds(16 * j, 16)], jnp.int32)
                    lo = plsc.bitcast(w << 16, jnp.float32)
                    hi = plsc.bitcast(w & HIMASK, jnp.float32)
                    vs += [lo, hi]
                return vs

            def dot(vs, ref, r):
                us = rowvecs(ref, r)
                acc = vs[0] * us[0]
                for j in range(1, len(vs)):
                    acc = acc + vs[j] * us[j]
                return jnp.sum(acc)

            def pos_group(g, carry):
                acc = jnp.zeros((LANES,), jnp.float32)
                for t in range(LANES):
                    i = g * LANES + t
                    vs = rowvecs(v_rows, i)
                    s = dot(vs, p_rows, i)
                    acc = jnp.where(lane == t, s, acc)
                pos_s[pl.ds(g * LANES, LANES)] = acc
                return carry

            lax.fori_loop(0, C // LANES, pos_group, 0)

            def neg_group(g, carry):
                accs = [jnp.zeros((LANES,), jnp.float32) for _ in range(5)]
                for ai in range(4):
                    i = g * 4 + ai
                    vs = rowvecs(v_rows, i)
                    for k in range(NEG):
                        rloc = ai * NEG + k
                        s = dot(vs, n_rows, i * NEG + k)
                        accs[rloc // LANES] = jnp.where(
                            lane == rloc % LANES, s, accs[rloc // LANES])
                for m in range(5):
                    neg_s[pl.ds(g * 4 * NEG + m * LANES, LANES)] = accs[m]
                return carry

            lax.fori_loop(0, C // 4, neg_group, 0)
            pltpu.sync_copy(pos_s, pos_out.at[pl.ds(off, C)])
            pltpu.sync_copy(neg_s, neg_out.at[pl.ds(off * NEG, NIDX)])
            return chunk_carry

        lax.fori_loop(0, NCHUNK, chunk, 0)

    return sc_scores


@functools.lru_cache(maxsize=None)
def _make_tc_relayout(V, D, CB):
    NB = (V + CB - 1) // CB
    Q = CB // 4

    def body(x_ref, y_ref):
        x = x_ref[...]
        lo = jax.lax.bitcast_convert_type(x[: D // 2, :], jnp.int32)
        hi = jax.lax.bitcast_convert_type(x[D // 2:, :], jnp.int32)
        word = (hi & HIMASK) | jax.lax.shift_right_logical(lo, 16)
        pk = jax.lax.bitcast_convert_type(word, jnp.float32).T
        y_ref[...] = jnp.concatenate(
            [pk[0 * Q:1 * Q], pk[1 * Q:2 * Q],
             pk[2 * Q:3 * Q], pk[3 * Q:4 * Q]], axis=-1)

    return pl.pallas_call(
        body,
        grid=(NB,),
        in_specs=[pl.BlockSpec((D, CB), lambda i: (0, i))],
        out_specs=pl.BlockSpec((Q, 2 * D), lambda i: (i, 0)),
        out_shape=jax.ShapeDtypeStruct((NB * Q, 2 * D), jnp.float32),
    )


def _log_sigmoid(x):
    return jnp.minimum(x, 0.0) - jnp.log1p(jnp.exp(-jnp.abs(x)))


@functools.lru_cache(maxsize=None)
def _make_tc_loss(B, NEG):
    def body(pos_ref, neg_ref, out_ref):
        pos_ls = _log_sigmoid(pos_ref[...])
        neg_ls = _log_sigmoid(-neg_ref[...])
        out_ref[0, 0] = -(jnp.sum(pos_ls) + jnp.sum(neg_ls)) / B

    return pl.pallas_call(
        body,
        out_shape=jax.ShapeDtypeStruct((1, 1), jnp.float32),
        out_specs=pl.BlockSpec(memory_space=pltpu.SMEM),
    )


def kernel(center_words, pos_context, neg_context, in_weight, out_weight):
    B, NEG = neg_context.shape
    V, D = in_weight.shape
    CB = 8192
    NB = (V + CB - 1) // CB

    def remap(idx):
        idx = idx.astype(jnp.int32)
        blk = idx // CB
        rem = idx % CB
        return blk * CB + (rem % (CB // 4)) * 4 + rem // (CB // 4)

    cw = remap(center_words)
    pc = remap(pos_context)
    ncr = remap(neg_context).reshape(B * NEG)
    relayout = _make_tc_relayout(V, D, CB)
    lin1 = relayout(in_weight.T).reshape(NB * CB, D // 2)
    lin2 = relayout(out_weight.T).reshape(NB * CB, D // 2)
    pos_s, neg_s = _make_sc_scores(B, NEG, D, 32)(
        cw, pc, ncr, lin1, lin2)
    loss = _make_tc_loss(B, NEG)(
        pos_s.reshape(B // 128, 128), neg_s.reshape(B * NEG // 128, 128))
    return loss.reshape(())


# speedup vs baseline: 1.4909x; 1.0334x over previous
import functools

import jax
import jax.numpy as jnp
from jax import lax
from jax.experimental import pallas as pl
from jax.experimental.pallas import tpu as pltpu
from jax.experimental.pallas import tpu_sc as plsc

NC = 2
NS = 16
LANES = 16
HIMASK = -65536


@functools.lru_cache(maxsize=None)
def _make_sc_scores(B, NEG, D, C):
    NW = NC * NS
    BPW = B // NW
    NCHUNK = BPW // C
    NIDX = C * NEG
    KROWS = NIDX // 128
    DP = D // 2

    mesh = plsc.VectorSubcoreMesh(core_axis_name="c", subcore_axis_name="s")

    @functools.partial(
        pl.kernel,
        mesh=mesh,
        compiler_params=pltpu.CompilerParams(
            needs_layout_passes=False, use_tc_tiling_on_sc=False),
        out_type=[
            jax.ShapeDtypeStruct((B,), jnp.float32),
            jax.ShapeDtypeStruct((B * NEG,), jnp.float32),
        ],
        scratch_types=[
            pltpu.VMEM((C,), jnp.int32),
            pltpu.VMEM((C,), jnp.int32),
            pltpu.VMEM((NIDX,), jnp.int32),
            pltpu.VMEM((C, DP), jnp.float32),
            pltpu.VMEM((C, DP), jnp.float32),
            pltpu.VMEM((NIDX, DP), jnp.float32),
            pltpu.VMEM((C,), jnp.float32),
            pltpu.VMEM((NIDX,), jnp.float32),
            pltpu.SemaphoreType.DMA,
        ],
    )
    def sc_scores(center_hbm, pos_hbm, negr_hbm, inw_hbm, outw_hbm,
                  pos_out, neg_out,
                  idx_c, idx_p, idx_n, v_rows, p_rows, n_rows,
                  pos_s, neg_s, sem):
        wid = lax.axis_index("s") * NC + lax.axis_index("c")
        base = wid * BPW

        def chunk(ci, chunk_carry):
            off = base + ci * C
            pltpu.sync_copy(center_hbm.at[pl.ds(off, C)], idx_c)
            pltpu.sync_copy(pos_hbm.at[pl.ds(off, C)], idx_p)
            pltpu.sync_copy(negr_hbm.at[pl.ds(off * NEG, NIDX)], idx_n)
            cps = [
                pltpu.async_copy(inw_hbm.at[idx_c], v_rows, sem),
                pltpu.async_copy(outw_hbm.at[idx_p], p_rows, sem),
            ]
            for j in range(KROWS):
                cps.append(pltpu.async_copy(
                    outw_hbm.at[idx_n.at[pl.ds(j * 128, 128)]],
                    n_rows.at[pl.ds(j * 128, 128)], sem))
            for cp in cps:
                cp.wait()

            lane = lax.iota(jnp.int32, LANES)

            def rowvecs(ref, r):
                vs = []
                for j in range(DP // 16):
                    w = plsc.bitcast(ref[r, pl.ds(16 * j, 16)], jnp.int32)
                    lo = plsc.bitcast(w << 16, jnp.float32)
                    hi = plsc.bitcast(w & HIMASK, jnp.float32)
                    vs += [lo, hi]
                return vs

            def dot(vs, ref, r):
                us = rowvecs(ref, r)
                acc = vs[0] * us[0]
                for j in range(1, len(vs)):
                    acc = acc + vs[j] * us[j]
                return jnp.sum(acc)

            def pos_group(g, carry):
                acc = jnp.zeros((LANES,), jnp.float32)
                for t in range(LANES):
                    i = g * LANES + t
                    vs = rowvecs(v_rows, i)
                    s = dot(vs, p_rows, i)
                    acc = jnp.where(lane == t, s, acc)
                pos_s[pl.ds(g * LANES, LANES)] = acc
                return carry

            lax.fori_loop(0, C // LANES, pos_group, 0)

            def neg_group(g, carry):
                accs = [jnp.zeros((LANES,), jnp.float32) for _ in range(5)]
                for ai in range(4):
                    i = g * 4 + ai
                    vs = rowvecs(v_rows, i)
                    for k in range(NEG):
                        rloc = ai * NEG + k
                        s = dot(vs, n_rows, i * NEG + k)
                        accs[rloc // LANES] = jnp.where(
                            lane == rloc % LANES, s, accs[rloc // LANES])
                for m in range(5):
                    neg_s[pl.ds(g * 4 * NEG + m * LANES, LANES)] = accs[m]
                return carry

            lax.fori_loop(0, C // 4, neg_group, 0)
            pltpu.sync_copy(pos_s, pos_out.at[pl.ds(off, C)])
            pltpu.sync_copy(neg_s, neg_out.at[pl.ds(off * NEG, NIDX)])
            return chunk_carry

        lax.fori_loop(0, NCHUNK, chunk, 0)

    return sc_scores


@functools.lru_cache(maxsize=None)
def _make_tc_relayout(V, D, CB):
    NB = (V + CB - 1) // CB
    Q = CB // 4

    def body(x_ref, y_ref):
        x = x_ref[...]
        lo = jax.lax.bitcast_convert_type(x[: D // 2, :], jnp.int32)
        hi = jax.lax.bitcast_convert_type(x[D // 2:, :], jnp.int32)
        word = (hi & HIMASK) | jax.lax.shift_right_logical(lo, 16)
        pk = jax.lax.bitcast_convert_type(word, jnp.float32).T
        y_ref[...] = jnp.concatenate(
            [pk[0 * Q:1 * Q], pk[1 * Q:2 * Q],
             pk[2 * Q:3 * Q], pk[3 * Q:4 * Q]], axis=-1)

    return pl.pallas_call(
        body,
        grid=(NB,),
        in_specs=[pl.BlockSpec((D, CB), lambda i: (0, i))],
        out_specs=pl.BlockSpec((Q, 2 * D), lambda i: (i, 0)),
        out_shape=jax.ShapeDtypeStruct((NB * Q, 2 * D), jnp.float32),
    )


def _log_sigmoid(x):
    return jnp.minimum(x, 0.0) - jnp.log1p(jnp.exp(-jnp.abs(x)))


@functools.lru_cache(maxsize=None)
def _make_tc_loss(B, NEG):
    def body(pos_ref, neg_ref, out_ref):
        pos_ls = _log_sigmoid(pos_ref[...])
        neg_ls = _log_sigmoid(-neg_ref[...])
        out_ref[0, 0] = -(jnp.sum(pos_ls) + jnp.sum(neg_ls)) / B

    return pl.pallas_call(
        body,
        out_shape=jax.ShapeDtypeStruct((1, 1), jnp.float32),
        out_specs=pl.BlockSpec(memory_space=pltpu.SMEM),
    )


def kernel(center_words, pos_context, neg_context, in_weight, out_weight):
    B, NEG = neg_context.shape
    V, D = in_weight.shape
    CB = 16384
    NB = (V + CB - 1) // CB

    def remap(idx):
        idx = idx.astype(jnp.int32)
        blk = idx // CB
        rem = idx % CB
        return blk * CB + (rem % (CB // 4)) * 4 + rem // (CB // 4)

    cw = remap(center_words)
    pc = remap(pos_context)
    ncr = remap(neg_context).reshape(B * NEG)
    relayout = _make_tc_relayout(V, D, CB)
    lin1 = relayout(in_weight.T).reshape(NB * CB, D // 2)
    lin2 = relayout(out_weight.T).reshape(NB * CB, D // 2)
    pos_s, neg_s = _make_sc_scores(B, NEG, D, 32)(
        cw, pc, ncr, lin1, lin2)
    loss = _make_tc_loss(B, NEG)(
        pos_s.reshape(B // 128, 128), neg_s.reshape(B * NEG // 128, 128))
    return loss.reshape(())
